# baseline (speedup 1.0000x reference)
import functools

import jax
import jax.numpy as jnp
from jax import lax
from jax.experimental import pallas as pl
from jax.experimental.pallas import tpu as pltpu
from jax.experimental.pallas import tpu_sc as plsc

NUM_SAMPLES_ = 32
SUBROWS = 16
BATCH = 8


def _subsample_rows(n_rows, row_elems, rows_per_worker):
    mesh = plsc.VectorSubcoreMesh(core_axis_name="c", subcore_axis_name="s")
    info = plsc.get_sparse_core_info()
    nc = info.num_cores

    n_batches = rows_per_worker // BATCH

    @functools.partial(
        pl.kernel,
        mesh=mesh,
        out_type=jax.ShapeDtypeStruct((n_rows, row_elems), jnp.float32),
        scratch_types=[
            pltpu.VMEM((rows_per_worker,), jnp.int32),
            pltpu.VMEM((BATCH, row_elems), jnp.float32),
            pltpu.SemaphoreType.DMA,
        ],
    )
    def k(src_hbm, x_hbm, out_hbm, idx_v, buf, sem):
        wid = lax.axis_index("s") * nc + lax.axis_index("c")
        base = wid * rows_per_worker
        pltpu.sync_copy(src_hbm.at[pl.ds(base, rows_per_worker)], idx_v)
        for b in range(n_batches):
            pltpu.async_copy(
                x_hbm.at[idx_v.at[pl.ds(b * BATCH, BATCH)]], buf, sem
            ).wait()
            pltpu.sync_copy(buf, out_hbm.at[pl.ds(base + b * BATCH, BATCH)])

    return k


def kernel(x):
    c, t, h, w = x.shape
    ns = NUM_SAMPLES_

    tf = jnp.clip(jnp.linspace(0.0, float(t - 1), ns), 0.0, float(t - 1))
    t_idx = tf.astype(jnp.int32)

    row_elems = (h * w) // SUBROWS
    n_out_rows = c * ns * SUBROWS

    sl = jnp.arange(c * ns, dtype=jnp.int32)
    src_slice = (sl // ns) * t + t_idx[sl % ns]
    q = jnp.arange(n_out_rows, dtype=jnp.int32)
    src_rows = src_slice[q // SUBROWS] * SUBROWS + (q % SUBROWS)

    x2 = x.reshape(c * t * SUBROWS, row_elems)
    rows_per_worker = n_out_rows // 32
    out = _subsample_rows(n_out_rows, row_elems, rows_per_worker)(src_rows, x2)
    return out.reshape(c, ns, h, w)


# speedup vs baseline: 1.0185x; 1.0185x over previous
import functools

import jax
import jax.numpy as jnp
from jax import lax
from jax.experimental import pallas as pl
from jax.experimental.pallas import tpu as pltpu
from jax.experimental.pallas import tpu_sc as plsc

NUM_SAMPLES_ = 32
SUBROWS = 16
BATCH = 8


NBUF = 3


def _subsample_rows(n_rows, row_elems, rows_per_worker):
    mesh = plsc.VectorSubcoreMesh(core_axis_name="c", subcore_axis_name="s")
    info = plsc.get_sparse_core_info()
    nc = info.num_cores

    n_batches = rows_per_worker // BATCH

    @functools.partial(
        pl.kernel,
        mesh=mesh,
        out_type=jax.ShapeDtypeStruct((n_rows, row_elems), jnp.float32),
        scratch_types=[
            pltpu.VMEM((rows_per_worker,), jnp.int32),
            pltpu.VMEM((NBUF, BATCH, row_elems), jnp.float32),
            pltpu.SemaphoreType.DMA((NBUF,)),
            pltpu.SemaphoreType.DMA((NBUF,)),
        ],
    )
    def k(src_hbm, x_hbm, out_hbm, idx_v, bufs, sem_g, sem_s):
        wid = lax.axis_index("s") * nc + lax.axis_index("c")
        base = wid * rows_per_worker
        pltpu.sync_copy(src_hbm.at[pl.ds(base, rows_per_worker)], idx_v)

        gathers = {}
        stores = {}

        def start_gather(b):
            i = b % NBUF
            gathers[b] = pltpu.async_copy(
                x_hbm.at[idx_v.at[pl.ds(b * BATCH, BATCH)]],
                bufs.at[i],
                sem_g.at[i],
            )

        def start_store(b):
            i = b % NBUF
            stores[b] = pltpu.async_copy(
                bufs.at[i],
                out_hbm.at[pl.ds(base + b * BATCH, BATCH)],
                sem_s.at[i],
            )

        for b in range(n_batches + NBUF - 1):
            if b < n_batches:
                if b >= NBUF:
                    stores.pop(b - NBUF).wait()
                start_gather(b)
            if b >= NBUF - 1:
                bb = b - (NBUF - 1)
                gathers.pop(bb).wait()
                start_store(bb)
        for bb in range(max(0, n_batches - NBUF), n_batches):
            if bb in stores:
                stores.pop(bb).wait()

    return k


def kernel(x):
    c, t, h, w = x.shape
    ns = NUM_SAMPLES_

    tf = jnp.clip(jnp.linspace(0.0, float(t - 1), ns), 0.0, float(t - 1))
    t_idx = tf.astype(jnp.int32)

    row_elems = (h * w) // SUBROWS
    n_out_rows = c * ns * SUBROWS

    sl = jnp.arange(c * ns, dtype=jnp.int32)
    src_slice = (sl // ns) * t + t_idx[sl % ns]
    q = jnp.arange(n_out_rows, dtype=jnp.int32)
    src_rows = src_slice[q // SUBROWS] * SUBROWS + (q % SUBROWS)

    x2 = x.reshape(c * t * SUBROWS, row_elems)
    rows_per_worker = n_out_rows // 32
    out = _subsample_rows(n_out_rows, row_elems, rows_per_worker)(src_rows, x2)
    return out.reshape(c, ns, h, w)


# speedup vs baseline: 4.9602x; 4.8700x over previous
import functools

import jax
import jax.numpy as jnp
from jax import lax
from jax.experimental import pallas as pl
from jax.experimental.pallas import tpu as pltpu
from jax.experimental.pallas import tpu_sc as plsc

NUM_SAMPLES_ = 32
NBUF = 2


def _subsample_slices(c, t, h, w, ns):
    mesh = plsc.VectorSubcoreMesh(core_axis_name="c", subcore_axis_name="s")
    info = plsc.get_sparse_core_info()
    nc = info.num_cores
    nw = nc * info.num_subcores

    half = h // 2
    slices_per_worker = (c * ns) // nw
    n_chunks = slices_per_worker * 2

    @functools.partial(
        pl.kernel,
        mesh=mesh,
        out_type=jax.ShapeDtypeStruct((c * ns, h, w), jnp.float32),
        scratch_types=[
            pltpu.VMEM((NBUF, half, w), jnp.float32),
            pltpu.SemaphoreType.DMA((NBUF,)),
            pltpu.SemaphoreType.DMA((NBUF,)),
        ],
    )
    def k(x_hbm, out_hbm, bufs, sem_g, sem_s):
        wid = lax.axis_index("s") * nc + lax.axis_index("c")
        r0 = wid * slices_per_worker

        def chunk(q):
            kk, hh = divmod(q, 2)
            r = r0 + kk
            cc = r // ns
            j = r % ns
            s = cc * t + (j * (t - 1)) // (ns - 1)
            return r, s, hh * half

        gathers = {}
        stores = {}

        def start_gather(q):
            i = q % NBUF
            _, s, h0 = chunk(q)
            gathers[q] = pltpu.async_copy(
                x_hbm.at[s, pl.ds(h0, half)], bufs.at[i], sem_g.at[i]
            )

        def start_store(q):
            i = q % NBUF
            r, _, h0 = chunk(q)
            stores[q] = pltpu.async_copy(
                bufs.at[i], out_hbm.at[r, pl.ds(h0, half)], sem_s.at[i]
            )

        for q in range(n_chunks + NBUF - 1):
            if q < n_chunks:
                if q >= NBUF:
                    stores.pop(q - NBUF).wait()
                start_gather(q)
            if q >= NBUF - 1:
                qq = q - (NBUF - 1)
                gathers.pop(qq).wait()
                start_store(qq)
        for qq in sorted(stores):
            stores.pop(qq).wait()

    return k


def kernel(x):
    c, t, h, w = x.shape
    ns = NUM_SAMPLES_
    x3 = x.reshape(c * t, h, w)
    out3 = _subsample_slices(c, t, h, w, ns)(x3)
    return out3.reshape(c, ns, h, w)


# speedup vs baseline: 5.0381x; 1.0157x over previous
import functools

import jax
import jax.numpy as jnp
from jax import lax
from jax.experimental import pallas as pl
from jax.experimental.pallas import tpu as pltpu
from jax.experimental.pallas import tpu_sc as plsc

NUM_SAMPLES_ = 32
NBUF = 3


def _subsample_slices(c, t, h, w, ns):
    mesh = plsc.VectorSubcoreMesh(core_axis_name="c", subcore_axis_name="s")
    info = plsc.get_sparse_core_info()
    nc = info.num_cores
    nw = nc * info.num_subcores

    half = h // 2
    slices_per_worker = (c * ns) // nw
    n_chunks = slices_per_worker * 2

    @functools.partial(
        pl.kernel,
        mesh=mesh,
        out_type=jax.ShapeDtypeStruct((c * ns, h, w), jnp.float32),
        scratch_types=[
            pltpu.VMEM((NBUF, half, w), jnp.float32),
            pltpu.SemaphoreType.DMA((NBUF,)),
            pltpu.SemaphoreType.DMA((NBUF,)),
        ],
    )
    def k(x_hbm, out_hbm, bufs, sem_g, sem_s):
        wid = lax.axis_index("s") * nc + lax.axis_index("c")
        r0 = wid * slices_per_worker

        def chunk(q):
            kk, hh = divmod(q, 2)
            r = r0 + kk
            cc = r // ns
            j = r % ns
            s = cc * t + (j * (t - 1)) // (ns - 1)
            return r, s, hh * half

        gathers = {}
        stores = {}

        def start_gather(q):
            i = q % NBUF
            _, s, h0 = chunk(q)
            gathers[q] = pltpu.async_copy(
                x_hbm.at[s, pl.ds(h0, half)], bufs.at[i], sem_g.at[i]
            )

        def start_store(q):
            i = q % NBUF
            r, _, h0 = chunk(q)
            stores[q] = pltpu.async_copy(
                bufs.at[i], out_hbm.at[r, pl.ds(h0, half)], sem_s.at[i]
            )

        for q in range(n_chunks + NBUF - 1):
            if q < n_chunks:
                if q >= NBUF:
                    stores.pop(q - NBUF).wait()
                start_gather(q)
            if q >= NBUF - 1:
                qq = q - (NBUF - 1)
                gathers.pop(qq).wait()
                start_store(qq)
        for qq in sorted(stores):
            stores.pop(qq).wait()

    return k


def kernel(x):
    c, t, h, w = x.shape
    ns = NUM_SAMPLES_
    x3 = x.reshape(c * t, h, w)
    out3 = _subsample_slices(c, t, h, w, ns)(x3)
    return out3.reshape(c, ns, h, w)


# speedup vs baseline: 5.0721x; 1.0067x over previous
import functools

import jax
import jax.numpy as jnp
from jax import lax
from jax.experimental import pallas as pl
from jax.experimental.pallas import tpu as pltpu
from jax.experimental.pallas import tpu_sc as plsc

NUM_SAMPLES_ = 32
NBUF = 6
CHUNK_H = 64


def _subsample_slices(c, t, h, w, ns):
    mesh = plsc.VectorSubcoreMesh(core_axis_name="c", subcore_axis_name="s")
    info = plsc.get_sparse_core_info()
    nc = info.num_cores
    nw = nc * info.num_subcores

    half = CHUNK_H
    per_slice = h // half
    slices_per_worker = (c * ns) // nw
    n_chunks = slices_per_worker * per_slice

    @functools.partial(
        pl.kernel,
        mesh=mesh,
        out_type=jax.ShapeDtypeStruct((c * ns, h, w), jnp.float32),
        scratch_types=[
            pltpu.VMEM((NBUF, half, w), jnp.float32),
            pltpu.SemaphoreType.DMA((NBUF,)),
            pltpu.SemaphoreType.DMA((NBUF,)),
        ],
    )
    def k(x_hbm, out_hbm, bufs, sem_g, sem_s):
        wid = lax.axis_index("s") * nc + lax.axis_index("c")
        r0 = wid * slices_per_worker

        def chunk(q):
            kk, hh = divmod(q, per_slice)
            r = r0 + kk
            cc = r // ns
            j = r % ns
            s = cc * t + (j * (t - 1)) // (ns - 1)
            return r, s, hh * half

        gathers = {}
        stores = {}

        def start_gather(q):
            i = q % NBUF
            _, s, h0 = chunk(q)
            gathers[q] = pltpu.async_copy(
                x_hbm.at[s, pl.ds(h0, half)], bufs.at[i], sem_g.at[i]
            )

        def start_store(q):
            i = q % NBUF
            r, _, h0 = chunk(q)
            stores[q] = pltpu.async_copy(
                bufs.at[i], out_hbm.at[r, pl.ds(h0, half)], sem_s.at[i]
            )

        for q in range(n_chunks + NBUF - 1):
            if q < n_chunks:
                if q >= NBUF:
                    stores.pop(q - NBUF).wait()
                start_gather(q)
            if q >= NBUF - 1:
                qq = q - (NBUF - 1)
                gathers.pop(qq).wait()
                start_store(qq)
        for qq in sorted(stores):
            stores.pop(qq).wait()

    return k


def kernel(x):
    c, t, h, w = x.shape
    ns = NUM_SAMPLES_
    x3 = x.reshape(c * t, h, w)
    out3 = _subsample_slices(c, t, h, w, ns)(x3)
    return out3.reshape(c, ns, h, w)
